# Initial kernel scaffold; baseline (speedup 1.0000x reference)
import jax
import jax.numpy as jnp
from jax.experimental import pallas as pl


def kernel(input, seqlengths, num_src, num_dst, num_pdf, num_weight, num_start, num_final, den_src, den_dst, den_pdf, den_weight, den_start, den_final):
    raise NotImplementedError("write your pallas kernel here")


# speedup vs baseline: 21.8472x; 21.8472x over previous
import functools

import jax
import jax.numpy as jnp
from jax import lax
from jax.experimental import pallas as pl
from jax.experimental.pallas import tpu as pltpu
from jax.experimental.pallas import tpu_sc as plsc

B = 16
T = 300
C = 2048
S_PAD = 2048
E_DEN = 16000
E_NUM = 1600
T_PAD = 304
NEG = -1e5
L = 16


def _sc_forward_body(ll_h, seql_h, src_h, dst_h, pdf_h, w_h, starts_h,
                     ealpha_o, scales_o,
                     src_v, dst_v, pdf_v, w_v, ealpha, enew, llbuf,
                     seql_v, scal_v):
    f = lax.axis_index("c")
    b = lax.axis_index("s")
    wid = f * 16 + b
    n_arc_chunks = jnp.where(f == 0, E_DEN // L, E_NUM // L)

    pltpu.sync_copy(src_h.at[f], src_v)
    pltpu.sync_copy(dst_h.at[f], dst_v)
    pltpu.sync_copy(pdf_h.at[f], pdf_v)
    pltpu.sync_copy(w_h.at[f], w_v)
    pltpu.sync_copy(starts_h.at[f], ealpha)
    pltpu.sync_copy(seql_h, seql_v)

    lanes = lax.iota(jnp.int32, L)
    seql_vec = seql_v[...]
    seqlen = seql_vec.at[jnp.full((L,), b, jnp.int32)].get(
        mode="promise_in_bounds")[0]

    def _expw(i, carry):
        sl = pl.ds(i * L, L)
        w_v[sl] = jnp.exp(w_v[sl])
        return carry
    lax.fori_loop(0, n_arc_chunks, _expw, 0)

    def _exps(i, carry):
        sl = pl.ds(i * L, L)
        ealpha[sl] = jnp.exp(ealpha[sl])
        return carry
    lax.fori_loop(0, S_PAD // L, _exps, 0)

    def _ones(i, carry):
        scal_v[pl.ds(i * L, L)] = jnp.full((L,), 1.0, jnp.float32)
        return carry
    lax.fori_loop(0, T_PAD // L, _ones, 0)

    def _step(t, carry):
        pltpu.sync_copy(ll_h.at[b, t], llbuf)

        def _expll(i, c2):
            sl = pl.ds(i * L, L)
            v = llbuf[sl]
            llbuf[sl] = jnp.exp(jnp.clip(v, -30.0, 30.0))
            return c2
        lax.fori_loop(0, C // L, _expll, 0)

        def _zero(i, c2):
            enew[pl.ds(i * L, L)] = jnp.zeros((L,), jnp.float32)
            return c2
        lax.fori_loop(0, S_PAD // L, _zero, 0)

        def _arcs(i, c2):
            sl = pl.ds(i * L, L)
            ea = plsc.load_gather(ealpha, [src_v[sl]])
            el = plsc.load_gather(llbuf, [pdf_v[sl]])
            contrib = ea * w_v[sl] * el
            plsc.addupdate_scatter(enew, [dst_v[sl]], contrib)
            return c2
        lax.fori_loop(0, n_arc_chunks, _arcs, 0)

        def _mx(i, acc):
            return jnp.maximum(acc, enew[pl.ds(i * L, L)])
        accv = lax.fori_loop(0, S_PAD // L, _mx,
                             jnp.zeros((L,), jnp.float32))
        for sh in (8, 4, 2, 1):
            perm = jnp.bitwise_and(lanes + sh, L - 1)
            accv = jnp.maximum(
                accv, accv.at[perm].get(mode="promise_in_bounds"))
        mbits = lax.bitcast_convert_type(accv, jnp.int32)
        ebits = jnp.bitwise_and(mbits, 0x7F800000)
        s_vec = lax.bitcast_convert_type(ebits, jnp.float32)
        inv_vec = lax.bitcast_convert_type(0x7F000000 - ebits, jnp.float32)
        plsc.store_scatter(scal_v, [jnp.full((L,), t, jnp.int32)],
                           s_vec, mask=lanes == 0)

        def _rs(i, c2):
            sl = pl.ds(i * L, L)
            ealpha[sl] = enew[sl] * inv_vec
            return c2
        lax.fori_loop(0, S_PAD // L, _rs, 0)
        return carry

    lax.fori_loop(0, seqlen, _step, 0)

    pltpu.sync_copy(ealpha, ealpha_o.at[wid])
    pltpu.sync_copy(scal_v, scales_o.at[wid])


def _tc_combine_body(scales_ref, ealpha_ref, finals_ref, out_ref):
    logs = jnp.log(scales_ref[...])
    acc = jnp.sum(logs, axis=1, keepdims=True)
    ef = jnp.exp(finals_ref[...])
    ef_rows = jnp.concatenate(
        [jnp.broadcast_to(ef[0:1, :], (16, S_PAD)),
         jnp.broadcast_to(ef[1:2, :], (16, S_PAD))], axis=0)
    mass = jnp.sum(ealpha_ref[...] * ef_rows, axis=1, keepdims=True)
    llh = acc + jnp.log(mass + 1e-30)
    sign = jnp.where(
        lax.broadcasted_iota(jnp.int32, (32, 1), 0) < 16, 1.0, -1.0)
    out_ref[...] = jnp.broadcast_to(jnp.sum(llh * sign), (1, 1))


def kernel(input, seqlengths, num_src, num_dst, num_pdf, num_weight,
           num_start, num_final, den_src, den_dst, den_pdf, den_weight,
           den_start, den_final):
    def pad_e(x, fill):
        return jnp.pad(x, (0, E_DEN - E_NUM), constant_values=fill)

    src2 = jnp.stack([den_src, pad_e(num_src, 0)])
    dst2 = jnp.stack([den_dst, pad_e(num_dst, 0)])
    pdf2 = jnp.stack([den_pdf, pad_e(num_pdf, 0)])
    w2 = jnp.stack([den_weight, pad_e(num_weight, NEG)])
    starts2 = jnp.stack([
        jnp.pad(den_start, (0, S_PAD - den_start.shape[0]),
                constant_values=NEG),
        jnp.pad(num_start, (0, S_PAD - num_start.shape[0]),
                constant_values=NEG)])
    finals2 = jnp.stack([
        jnp.pad(den_final, (0, S_PAD - den_final.shape[0]),
                constant_values=NEG),
        jnp.pad(num_final, (0, S_PAD - num_final.shape[0]),
                constant_values=NEG)])

    mesh = plsc.VectorSubcoreMesh(core_axis_name="c", subcore_axis_name="s")
    sc_fwd = pl.kernel(
        _sc_forward_body,
        out_type=(jax.ShapeDtypeStruct((32, S_PAD), jnp.float32),
                  jax.ShapeDtypeStruct((32, T_PAD), jnp.float32)),
        mesh=mesh,
        compiler_params=pltpu.CompilerParams(needs_layout_passes=False),
        scratch_types=[
            pltpu.VMEM((E_DEN,), jnp.int32),
            pltpu.VMEM((E_DEN,), jnp.int32),
            pltpu.VMEM((E_DEN,), jnp.int32),
            pltpu.VMEM((E_DEN,), jnp.float32),
            pltpu.VMEM((S_PAD,), jnp.float32),
            pltpu.VMEM((S_PAD,), jnp.float32),
            pltpu.VMEM((C,), jnp.float32),
            pltpu.VMEM((L,), jnp.int32),
            pltpu.VMEM((T_PAD,), jnp.float32),
        ],
    )
    ealpha32, scales32 = sc_fwd(input, seqlengths, src2, dst2, pdf2, w2,
                                starts2)

    loss11 = pl.pallas_call(
        _tc_combine_body,
        out_shape=jax.ShapeDtypeStruct((1, 1), jnp.float32),
    )(scales32, ealpha32, finals2)
    return loss11[0, 0]


# speedup vs baseline: 67.5731x; 3.0930x over previous
import jax
import jax.numpy as jnp
from jax import lax
from jax.experimental import pallas as pl
from jax.experimental.pallas import tpu as pltpu
from jax.experimental.pallas import tpu_sc as plsc

B = 16
T = 300
C = 2048
S_PAD = 2048
E_DEN = 16000
E_NUM = 1600
T_PAD = 304
NEG = -1e5
L = 16


def _sc_forward_body(ll_h, seql_h, sd_h, pdf_h, w_h, starts_h,
                     ealpha_o, scales_o,
                     sd_v, pdf_v, w_v, bufA, bufB, llA, llB,
                     seql_v, scal_v, inv_v, semA, semB):
    f = lax.axis_index("c")
    b = lax.axis_index("s")
    wid = f * 16 + b
    is_den = f == 0
    lanes = lax.iota(jnp.int32, L)

    pltpu.sync_copy(sd_h.at[f], sd_v)
    pltpu.sync_copy(pdf_h.at[f], pdf_v)
    pltpu.sync_copy(w_h.at[f], w_v)
    pltpu.sync_copy(starts_h.at[f], bufA)
    pltpu.sync_copy(seql_h, seql_v)
    seqlen = seql_v[...].at[jnp.full((L,), b, jnp.int32)].get(
        mode="promise_in_bounds")[0]

    @plsc.parallel_loop(0, E_DEN // L, unroll=4)
    def _expw(i):
        sl = pl.ds(i * L, L)
        w_v[sl] = jnp.exp(w_v[sl])

    @plsc.parallel_loop(0, S_PAD // L, unroll=4)
    def _init(i):
        sl = pl.ds(i * L, L)
        bufA[sl] = jnp.exp(bufA[sl])
        bufB[sl] = jnp.zeros((L,), jnp.float32)

    @plsc.parallel_loop(0, T_PAD // L, unroll=4)
    def _ones(i):
        scal_v[pl.ds(i * L, L)] = jnp.full((L,), 1.0, jnp.float32)

    inv_v[...] = jnp.full((L,), 1.0, jnp.float32)
    pltpu.async_copy(ll_h.at[b, 0], llA, semA)

    def half_step(t, rbuf, wbuf, llbuf):
        ivec = inv_v[...]

        @plsc.parallel_loop(0, C // L, unroll=4)
        def _expll(i):
            sl = pl.ds(i * L, L)
            v = llbuf[sl]
            llbuf[sl] = jnp.exp(jnp.clip(v, -30.0, 30.0)) * ivec

        def arc_pass(n_chunks, unroll):
            @plsc.parallel_loop(0, n_chunks, unroll=unroll)
            def _arcs(i):
                sl = pl.ds(i * L, L)
                sd = sd_v[sl]
                src = jnp.bitwise_and(sd, 0xFFFF)
                dst = lax.shift_right_logical(sd, 16)
                ea = plsc.load_gather(rbuf, [src])
                el = plsc.load_gather(llbuf, [pdf_v[sl]])
                contrib = ea * w_v[sl] * el
                plsc.addupdate_scatter(wbuf, [dst], contrib)

        @pl.when(is_den)
        def _():
            arc_pass(E_DEN // L, 8)

        @pl.when(jnp.logical_not(is_den))
        def _():
            arc_pass(E_NUM // L, 4)

        @plsc.parallel_loop(0, S_PAD // L, unroll=4,
                            carry=jnp.zeros((L,), jnp.float32))
        def _mxz(i, acc):
            sl = pl.ds(i * L, L)
            acc = jnp.maximum(acc, wbuf[sl])
            rbuf[sl] = jnp.zeros((L,), jnp.float32)
            return acc
        accv = _mxz

        for sh in (8, 4, 2, 1):
            perm = jnp.bitwise_and(lanes + sh, L - 1)
            accv = jnp.maximum(
                accv, accv.at[perm].get(mode="promise_in_bounds"))
        ebits = jnp.bitwise_and(lax.bitcast_convert_type(accv, jnp.int32),
                                0x7F800000)
        s_vec = lax.bitcast_convert_type(ebits, jnp.float32)
        plsc.store_scatter(scal_v, [jnp.full((L,), t, jnp.int32)],
                           s_vec, mask=lanes == 0)
        inv_v[...] = lax.bitcast_convert_type(0x7F000000 - ebits,
                                              jnp.float32)

    def k_body(k, carry):
        t0 = 2 * k
        t1 = t0 + 1
        t2 = t0 + 2

        @pl.when(t1 < seqlen)
        def _():
            pltpu.async_copy(ll_h.at[b, t1], llB, semB)
        pltpu.make_async_copy(ll_h.at[b, 0], llA, semA).wait()
        half_step(t0, bufA, bufB, llA)

        @pl.when(t2 < seqlen)
        def _():
            pltpu.async_copy(ll_h.at[b, t2], llA, semA)

        @pl.when(t1 < seqlen)
        def _():
            pltpu.make_async_copy(ll_h.at[b, 0], llB, semB).wait()
            half_step(t1, bufB, bufA, llB)
        return carry

    lax.fori_loop(0, (seqlen + 1) >> 1, k_body, 0)

    fvec = inv_v[...]

    @pl.when(jnp.bitwise_and(seqlen, 1) == 0)
    def _():
        @plsc.parallel_loop(0, S_PAD // L, unroll=4)
        def _oA(i):
            sl = pl.ds(i * L, L)
            bufA[sl] = bufA[sl] * fvec

    @pl.when(jnp.bitwise_and(seqlen, 1) == 1)
    def _():
        @plsc.parallel_loop(0, S_PAD // L, unroll=4)
        def _oB(i):
            sl = pl.ds(i * L, L)
            bufA[sl] = bufB[sl] * fvec

    pltpu.sync_copy(bufA, ealpha_o.at[wid])
    pltpu.sync_copy(scal_v, scales_o.at[wid])


def _tc_combine_body(scales_ref, ealpha_ref, finals_ref, out_ref):
    logs = jnp.log(scales_ref[...])
    acc = jnp.sum(logs, axis=1, keepdims=True)
    ef = jnp.exp(finals_ref[...])
    ef_rows = jnp.concatenate(
        [jnp.broadcast_to(ef[0:1, :], (16, S_PAD)),
         jnp.broadcast_to(ef[1:2, :], (16, S_PAD))], axis=0)
    mass = jnp.sum(ealpha_ref[...] * ef_rows, axis=1, keepdims=True)
    llh = acc + jnp.log(mass + 1e-30)
    sign = jnp.where(
        lax.broadcasted_iota(jnp.int32, (32, 1), 0) < 16, 1.0, -1.0)
    out_ref[...] = jnp.broadcast_to(jnp.sum(llh * sign), (1, 1))


def kernel(input, seqlengths, num_src, num_dst, num_pdf, num_weight,
           num_start, num_final, den_src, den_dst, den_pdf, den_weight,
           den_start, den_final):
    def pad_e(x, fill):
        return jnp.pad(x, (0, E_DEN - E_NUM), constant_values=fill)

    sd2 = jnp.stack([den_src | (den_dst << 16),
                     pad_e(num_src | (num_dst << 16), 0)])
    pdf2 = jnp.stack([den_pdf, pad_e(num_pdf, 0)])
    w2 = jnp.stack([den_weight, pad_e(num_weight, NEG)])
    starts2 = jnp.stack([
        jnp.pad(den_start, (0, S_PAD - den_start.shape[0]),
                constant_values=NEG),
        jnp.pad(num_start, (0, S_PAD - num_start.shape[0]),
                constant_values=NEG)])
    finals2 = jnp.stack([
        jnp.pad(den_final, (0, S_PAD - den_final.shape[0]),
                constant_values=NEG),
        jnp.pad(num_final, (0, S_PAD - num_final.shape[0]),
                constant_values=NEG)])

    mesh = plsc.VectorSubcoreMesh(core_axis_name="c", subcore_axis_name="s")
    sc_fwd = pl.kernel(
        _sc_forward_body,
        out_type=(jax.ShapeDtypeStruct((32, S_PAD), jnp.float32),
                  jax.ShapeDtypeStruct((32, T_PAD), jnp.float32)),
        mesh=mesh,
        compiler_params=pltpu.CompilerParams(needs_layout_passes=False),
        scratch_types=[
            pltpu.VMEM((E_DEN,), jnp.int32),
            pltpu.VMEM((E_DEN,), jnp.int32),
            pltpu.VMEM((E_DEN,), jnp.float32),
            pltpu.VMEM((S_PAD,), jnp.float32),
            pltpu.VMEM((S_PAD,), jnp.float32),
            pltpu.VMEM((C,), jnp.float32),
            pltpu.VMEM((C,), jnp.float32),
            pltpu.VMEM((L,), jnp.int32),
            pltpu.VMEM((T_PAD,), jnp.float32),
            pltpu.VMEM((L,), jnp.float32),
            pltpu.SemaphoreType.DMA,
            pltpu.SemaphoreType.DMA,
        ],
    )
    ealpha32, scales32 = sc_fwd(input, seqlengths, sd2, pdf2, w2, starts2)

    loss11 = pl.pallas_call(
        _tc_combine_body,
        out_shape=jax.ShapeDtypeStruct((1, 1), jnp.float32),
    )(scales32, ealpha32, finals2)
    return loss11[0, 0]
